# baseline (speedup 1.0000x reference)
import jax
import jax.numpy as jnp
from jax import lax
from jax.experimental import pallas as pl
from jax.experimental.pallas import tpu as pltpu
from jax.experimental.pallas import tpu_sc as plsc

N = 10000
E = 320000
D_IN = 128
D_H = 128
D_OUT = 16

NC = 2
NS = 16
NW = NC * NS
EPW = E // NW
K = 80
NCH = EPW // K
NP = 10240
RPT = NP // NS

_MESH = dict(core_axis_name="c", subcore_axis_name="s",
             num_cores=NC, num_subcores=NS)


NBUF = 5


def _deg_body(ei_hbm, zeros_hbm, ones_hbm, degs_out, degd_out,
              idx_v, ones_v, degs_sh, degd_sh, *sems):
    cid = lax.axis_index("c")
    sid = lax.axis_index("s")
    wid = cid * NS + sid
    pltpu.sync_copy(zeros_hbm, degs_sh.at[pl.ds(sid * RPT, RPT), :])
    pltpu.sync_copy(zeros_hbm, degd_sh.at[pl.ds(sid * RPT, RPT), :])

    pltpu.sync_copy(ones_hbm, ones_v)
    pltpu.sync_copy(ei_hbm.at[wid], idx_v)
    plsc.subcore_barrier()

    def _g(g, c):
        for b in range(NBUF):
            j = g * NBUF + b
            pltpu.async_copy(ones_v, degs_sh.at[idx_v.at[j, 0]], sems[b],
                             add=True)
            pltpu.async_copy(ones_v, degd_sh.at[idx_v.at[j, 1]],
                             sems[NBUF + b], add=True)
        for b in range(NBUF):
            j = g * NBUF + b
            pltpu.make_async_copy(ones_v, degs_sh.at[idx_v.at[j, 0]],
                                  sems[b]).wait()
            pltpu.make_async_copy(ones_v, degd_sh.at[idx_v.at[j, 1]],
                                  sems[NBUF + b]).wait()
        return c
    lax.fori_loop(0, NCH // NBUF, _g, 0)

    plsc.subcore_barrier()
    pltpu.sync_copy(degs_sh.at[pl.ds(sid * RPT, RPT), :],
                    degs_out.at[cid, pl.ds(sid * RPT, RPT), :])
    pltpu.sync_copy(degd_sh.at[pl.ds(sid * RPT, RPT), :],
                    degd_out.at[cid, pl.ds(sid * RPT, RPT), :])


def _agg1_body(x_hbm, ei_hbm, zeros_hbm, out_hbm,
               idx_v, rows_v, acc_sh, *sems):
    isems = sems[:5]
    gsems = sems[5:]
    cid = lax.axis_index("c")
    sid = lax.axis_index("s")
    wid = cid * NS + sid
    pltpu.sync_copy(zeros_hbm, acc_sh.at[pl.ds(sid * RPT, RPT), :])
    plsc.subcore_barrier()

    def idx_load(j, b5):
        pltpu.async_copy(ei_hbm.at[wid, pl.ds(j, 1)], idx_v.at[b5], isems[b5])

    def idx_wait(j, b5):
        pltpu.make_async_copy(ei_hbm.at[wid, pl.ds(j, 1)], idx_v.at[b5],
                              isems[b5]).wait()

    def gather(g, b5, b3):
        idx_wait(g, b5)
        pltpu.async_copy(x_hbm.at[idx_v.at[b5, 0, 0]], rows_v.at[b3],
                         gsems[b3])

    def step(j, i, do_load, do_gather):
        b3, b5 = i % 3, i % 5
        pltpu.make_async_copy(x_hbm.at[idx_v.at[b5, 0, 0]], rows_v.at[b3],
                              gsems[b3]).wait()
        pltpu.sync_copy(rows_v.at[b3], acc_sh.at[idx_v.at[b5, 0, 1]],
                        add=True)
        if do_load:
            idx_load(j + 5, b5)
        if do_gather:
            gather(j + 3, (i + 3) % 5, (i + 3) % 3)

    for b in range(5):
        idx_load(b, b)
    for c in range(3):
        gather(c, c, c)

    def _t(t, c):
        for i in range(15):
            j = t * 15 + i
            step(j, i, True, True)
        return c
    lax.fori_loop(0, NCH // 15, _t, 0)
    for j in range(NCH // 15 * 15, NCH):
        step(j, j % 15, j + 5 < NCH, j + 3 < NCH)

    plsc.subcore_barrier()
    pltpu.sync_copy(acc_sh.at[pl.ds(sid * RPT, RPT), :],
                    out_hbm.at[cid, pl.ds(sid * RPT, RPT), :])


NB2 = 10


def _agg2_body(x_hbm, ei_hbm, zeros_hbm, out_hbm,
               idx_v, rows_v, acc_sh, *sems):
    cid = lax.axis_index("c")
    sid = lax.axis_index("s")
    wid = cid * NS + sid
    pltpu.sync_copy(zeros_hbm, acc_sh.at[pl.ds(sid * RPT, RPT), :])
    pltpu.sync_copy(ei_hbm.at[wid], idx_v)
    plsc.subcore_barrier()

    for b in range(NB2):
        pltpu.async_copy(x_hbm.at[idx_v.at[b, 0]], rows_v.at[b], sems[b])

    def step(j, b, do_gather):
        pltpu.make_async_copy(x_hbm.at[idx_v.at[j, 0]], rows_v.at[b],
                              sems[b]).wait()
        pltpu.sync_copy(rows_v.at[b], acc_sh.at[idx_v.at[j, 1]], add=True)
        if do_gather:
            pltpu.async_copy(x_hbm.at[idx_v.at[j + NB2, 0]], rows_v.at[b],
                             sems[b])

    def _g(g, c):
        for b in range(NB2):
            step(g * NB2 + b, b, True)
        return c
    nfull = NCH // NB2 - 1
    lax.fori_loop(0, nfull, _g, 0)
    for j in range(nfull * NB2, NCH):
        step(j, j % NB2, j + NB2 < NCH)

    plsc.subcore_barrier()
    pltpu.sync_copy(acc_sh.at[pl.ds(sid * RPT, RPT), :],
                    out_hbm.at[cid, pl.ds(sid * RPT, RPT), :])


def _prep_body(x_ref, ds_ref, dd_ref, xs_ref, ns_ref, nd_ref):
    ns = lax.rsqrt(jnp.maximum(ds_ref[0] + ds_ref[1], 1.0))
    nd = lax.rsqrt(jnp.maximum(dd_ref[0] + dd_ref[1], 1.0))
    xs_ref[...] = x_ref[...] * ns
    ns_ref[...] = ns
    nd_ref[...] = nd


def _mlp_body(a0_ref, nd_ref, ns_ref, w1_ref, b1_ref, w2_ref, out_ref):
    a = (a0_ref[0] + a0_ref[1]) * nd_ref[...]
    h = jnp.dot(a, w1_ref[...], preferred_element_type=jnp.float32)
    h = jnp.maximum(h + b1_ref[...], 0.0)
    out_ref[...] = jnp.dot(h * ns_ref[...], w2_ref[...], preferred_element_type=jnp.float32)


def _final_body(a_ref, nd_ref, b2_ref, out_ref):
    out_ref[...] = (a_ref[0] + a_ref[1]) * nd_ref[...] + b2_ref[...]


def kernel(x, edge_index, W1, b1, W2, b2):
    ei_t = edge_index.reshape(2, NW, NCH, K).transpose(1, 2, 0, 3)
    zeros_r = jnp.zeros((RPT, 1), jnp.float32)
    ones_c = jnp.ones((K, 1), jnp.float32)
    zeros_h = jnp.zeros((RPT, D_H), jnp.float32)
    zeros_o = jnp.zeros((RPT, D_OUT), jnp.float32)

    mesh = plsc.VectorSubcoreMesh(**_MESH)
    degs_p, degd_p = pl.kernel(
        _deg_body,
        out_type=(jax.ShapeDtypeStruct((NC, NP, 1), jnp.float32),
                  jax.ShapeDtypeStruct((NC, NP, 1), jnp.float32)),
        mesh=mesh,
        scratch_types=(
            pltpu.VMEM((NCH, 2, K), jnp.int32),
            pltpu.VMEM((K, 1), jnp.float32),
            pltpu.VMEM_SHARED((NP, 1), jnp.float32),
            pltpu.VMEM_SHARED((NP, 1), jnp.float32),
        ) + (pltpu.SemaphoreType.DMA,) * (2 * NBUF),
    )(ei_t, zeros_r, ones_c)

    B = 2000
    G = N // B
    degs3, degd3 = degs_p, degd_p
    xs, ns, nd = pl.pallas_call(
        _prep_body,
        grid=(G,),
        in_specs=[pl.BlockSpec((B, D_IN), lambda i: (i, 0)),
                  pl.BlockSpec((NC, B, 1), lambda i: (0, i, 0)),
                  pl.BlockSpec((NC, B, 1), lambda i: (0, i, 0))],
        out_specs=[pl.BlockSpec((B, D_IN), lambda i: (i, 0)),
                   pl.BlockSpec((B, 1), lambda i: (i, 0)),
                   pl.BlockSpec((B, 1), lambda i: (i, 0))],
        out_shape=[jax.ShapeDtypeStruct((N, D_IN), jnp.float32),
                   jax.ShapeDtypeStruct((N, 1), jnp.float32),
                   jax.ShapeDtypeStruct((N, 1), jnp.float32)],
    )(x, degs3, degd3)

    mesh = plsc.VectorSubcoreMesh(**_MESH)
    agg1 = pl.kernel(
        _agg1_body,
        out_type=jax.ShapeDtypeStruct((NC, NP, D_H), jnp.float32),
        mesh=mesh,
        scratch_types=(
            pltpu.VMEM((5, 1, 2, K), jnp.int32),
            pltpu.VMEM((3, K, D_H), jnp.float32),
            pltpu.VMEM_SHARED((NP, D_H), jnp.float32),
        ) + (pltpu.SemaphoreType.DMA,) * 8,
    )(xs, ei_t, zeros_h)

    b1r = b1.reshape(1, D_H)
    y2 = pl.pallas_call(
        _mlp_body,
        grid=(G,),
        in_specs=[pl.BlockSpec((NC, B, D_H), lambda i: (0, i, 0)),
                  pl.BlockSpec((B, 1), lambda i: (i, 0)),
                  pl.BlockSpec((B, 1), lambda i: (i, 0)),
                  pl.BlockSpec((D_H, D_H), lambda i: (0, 0)),
                  pl.BlockSpec((1, D_H), lambda i: (0, 0)),
                  pl.BlockSpec((D_H, D_OUT), lambda i: (0, 0))],
        out_specs=pl.BlockSpec((B, D_OUT), lambda i: (i, 0)),
        out_shape=jax.ShapeDtypeStruct((N, D_OUT), jnp.float32),
    )(agg1, nd, ns, W1, b1r, W2)

    mesh = plsc.VectorSubcoreMesh(**_MESH)
    agg2 = pl.kernel(
        _agg2_body,
        out_type=jax.ShapeDtypeStruct((NC, NP, D_OUT), jnp.float32),
        mesh=mesh,
        compiler_params=pltpu.CompilerParams(use_tc_tiling_on_sc=False),
        scratch_types=(
            pltpu.VMEM((NCH, 2, K), jnp.int32),
            pltpu.VMEM((NB2, K, D_OUT), jnp.float32),
            pltpu.VMEM_SHARED((NP, D_OUT), jnp.float32),
        ) + (pltpu.SemaphoreType.DMA,) * NB2,
    )(y2, ei_t, zeros_o)

    b2r = b2.reshape(1, D_OUT)
    out = pl.pallas_call(
        _final_body,
        grid=(G,),
        in_specs=[pl.BlockSpec((NC, B, D_OUT), lambda i: (0, i, 0)),
                  pl.BlockSpec((B, 1), lambda i: (i, 0)),
                  pl.BlockSpec((1, D_OUT), lambda i: (0, 0))],
        out_specs=pl.BlockSpec((B, D_OUT), lambda i: (i, 0)),
        out_shape=jax.ShapeDtypeStruct((N, D_OUT), jnp.float32),
    )(agg2, nd, b2r)
    return out


# speedup vs baseline: 1.0513x; 1.0513x over previous
import jax
import jax.numpy as jnp
from jax import lax
from jax.experimental import pallas as pl
from jax.experimental.pallas import tpu as pltpu
from jax.experimental.pallas import tpu_sc as plsc

N = 10000
E = 320000
D_IN = 128
D_H = 128
D_OUT = 16

NC = 2
NS = 16
NW = NC * NS
EPW = E // NW
K = 80
NCH = EPW // K
NP = 10240
RPT = NP // NS

_MESH = dict(core_axis_name="c", subcore_axis_name="s",
             num_cores=NC, num_subcores=NS)


NBUF = 5


def _deg_body(ei_hbm, zeros_hbm, ones_hbm, degs_out, degd_out,
              idx_v, ones_v, degs_sh, degd_sh, *sems):
    cid = lax.axis_index("c")
    sid = lax.axis_index("s")
    wid = cid * NS + sid
    pltpu.sync_copy(zeros_hbm, degs_sh.at[pl.ds(sid * RPT, RPT)])
    pltpu.sync_copy(zeros_hbm, degd_sh.at[pl.ds(sid * RPT, RPT)])
    pltpu.sync_copy(ones_hbm, ones_v)
    pltpu.sync_copy(ei_hbm.at[wid], idx_v)
    plsc.subcore_barrier()


    def _g(g, c):
        for b in range(NBUF):
            j = g * NBUF + b
            pltpu.async_copy(ones_v, degs_sh.at[idx_v.at[j, 0]], sems[b],
                             add=True)
            pltpu.async_copy(ones_v, degd_sh.at[idx_v.at[j, 1]],
                             sems[NBUF + b], add=True)
        for b in range(NBUF):
            j = g * NBUF + b
            pltpu.make_async_copy(ones_v, degs_sh.at[idx_v.at[j, 0]],
                                  sems[b]).wait()
            pltpu.make_async_copy(ones_v, degd_sh.at[idx_v.at[j, 1]],
                                  sems[NBUF + b]).wait()
        return c
    lax.fori_loop(0, NCH // NBUF, _g, 0)

    plsc.subcore_barrier()
    pltpu.sync_copy(degs_sh.at[pl.ds(sid * RPT, RPT)],
                    degs_out.at[cid, pl.ds(sid * RPT, RPT)])
    pltpu.sync_copy(degd_sh.at[pl.ds(sid * RPT, RPT)],
                    degd_out.at[cid, pl.ds(sid * RPT, RPT)])


def _agg1_body(x_hbm, ei_hbm, zeros_hbm, out_hbm,
               idx_v, rows_v, acc_sh, *sems):
    isems = sems[:5]
    gsems = sems[5:]
    cid = lax.axis_index("c")
    sid = lax.axis_index("s")
    wid = cid * NS + sid
    pltpu.sync_copy(zeros_hbm, acc_sh.at[pl.ds(sid * RPT, RPT), :])
    plsc.subcore_barrier()

    def idx_load(j, b5):
        pltpu.async_copy(ei_hbm.at[wid, pl.ds(j, 1)], idx_v.at[b5], isems[b5])

    def idx_wait(j, b5):
        pltpu.make_async_copy(ei_hbm.at[wid, pl.ds(j, 1)], idx_v.at[b5],
                              isems[b5]).wait()

    def gather(g, b5, b3):
        idx_wait(g, b5)
        pltpu.async_copy(x_hbm.at[idx_v.at[b5, 0, 0]], rows_v.at[b3],
                         gsems[b3])

    def step(j, i, do_load, do_gather):
        b3, b5 = i % 3, i % 5
        pltpu.make_async_copy(x_hbm.at[idx_v.at[b5, 0, 0]], rows_v.at[b3],
                              gsems[b3]).wait()
        pltpu.sync_copy(rows_v.at[b3], acc_sh.at[idx_v.at[b5, 0, 1]],
                        add=True)
        if do_load:
            idx_load(j + 5, b5)
        if do_gather:
            gather(j + 3, (i + 3) % 5, (i + 3) % 3)

    for b in range(5):
        idx_load(b, b)
    for c in range(3):
        gather(c, c, c)

    def _t(t, c):
        for i in range(15):
            j = t * 15 + i
            step(j, i, True, True)
        return c
    lax.fori_loop(0, NCH // 15, _t, 0)
    for j in range(NCH // 15 * 15, NCH):
        step(j, j % 15, j + 5 < NCH, j + 3 < NCH)

    plsc.subcore_barrier()
    pltpu.sync_copy(acc_sh.at[pl.ds(sid * RPT, RPT), :],
                    out_hbm.at[cid, pl.ds(sid * RPT, RPT), :])


NB2 = 10


def _agg2_body(x_hbm, ei_hbm, zeros_hbm, out_hbm,
               idx_v, rows_v, acc_sh, *sems):
    cid = lax.axis_index("c")
    sid = lax.axis_index("s")
    wid = cid * NS + sid
    pltpu.sync_copy(zeros_hbm, acc_sh.at[pl.ds(sid * RPT, RPT), :])
    pltpu.sync_copy(ei_hbm.at[wid], idx_v)
    plsc.subcore_barrier()

    for b in range(NB2):
        pltpu.async_copy(x_hbm.at[idx_v.at[b, 0]], rows_v.at[b], sems[b])

    def step(j, b, do_gather):
        pltpu.make_async_copy(x_hbm.at[idx_v.at[j, 0]], rows_v.at[b],
                              sems[b]).wait()
        pltpu.sync_copy(rows_v.at[b], acc_sh.at[idx_v.at[j, 1]], add=True)
        if do_gather:
            pltpu.async_copy(x_hbm.at[idx_v.at[j + NB2, 0]], rows_v.at[b],
                             sems[b])

    def _g(g, c):
        for b in range(NB2):
            step(g * NB2 + b, b, True)
        return c
    nfull = NCH // NB2 - 1
    lax.fori_loop(0, nfull, _g, 0)
    for j in range(nfull * NB2, NCH):
        step(j, j % NB2, j + NB2 < NCH)

    plsc.subcore_barrier()
    pltpu.sync_copy(acc_sh.at[pl.ds(sid * RPT, RPT), :],
                    out_hbm.at[cid, pl.ds(sid * RPT, RPT), :])


def _prep_body(x_ref, ds_ref, dd_ref, xs_ref, ns_ref, nd_ref):
    ns = lax.rsqrt(jnp.maximum(ds_ref[0] + ds_ref[1], 1.0))
    nd = lax.rsqrt(jnp.maximum(dd_ref[0] + dd_ref[1], 1.0))
    xs_ref[...] = x_ref[...] * ns
    ns_ref[...] = ns
    nd_ref[...] = nd


def _mlp_body(a0_ref, nd_ref, ns_ref, w1_ref, b1_ref, w2_ref, out_ref):
    a = (a0_ref[0] + a0_ref[1]) * nd_ref[...]
    h = jnp.dot(a, w1_ref[...], preferred_element_type=jnp.float32)
    h = jnp.maximum(h + b1_ref[...], 0.0)
    out_ref[...] = jnp.dot(h * ns_ref[...], w2_ref[...], preferred_element_type=jnp.float32)


def _final_body(a_ref, nd_ref, b2_ref, out_ref):
    out_ref[...] = (a_ref[0] + a_ref[1]) * nd_ref[...] + b2_ref[...]


def kernel(x, edge_index, W1, b1, W2, b2):
    ei_t = edge_index.reshape(2, NW, NCH, K).transpose(1, 2, 0, 3)
    zeros_r = jnp.zeros((RPT,), jnp.float32)
    ones_c = jnp.ones((K,), jnp.float32)
    zeros_h = jnp.zeros((RPT, D_H), jnp.float32)
    zeros_o = jnp.zeros((RPT, D_OUT), jnp.float32)

    mesh = plsc.VectorSubcoreMesh(**_MESH)
    degs_p, degd_p = pl.kernel(
        _deg_body,
        out_type=(jax.ShapeDtypeStruct((NC, NP), jnp.float32),
                  jax.ShapeDtypeStruct((NC, NP), jnp.float32)),
        mesh=mesh,
        scratch_types=(
            pltpu.VMEM((NCH, 2, K), jnp.int32),
            pltpu.VMEM((K,), jnp.float32),
            pltpu.VMEM_SHARED((NP,), jnp.float32),
            pltpu.VMEM_SHARED((NP,), jnp.float32),
        ) + (pltpu.SemaphoreType.DMA,) * (2 * NBUF),
    )(ei_t, zeros_r, ones_c)

    B = 2000
    G = N // B
    degs3 = degs_p.reshape(NC, NP, 1)
    degd3 = degd_p.reshape(NC, NP, 1)
    xs, ns, nd = pl.pallas_call(
        _prep_body,
        grid=(G,),
        in_specs=[pl.BlockSpec((B, D_IN), lambda i: (i, 0)),
                  pl.BlockSpec((NC, B, 1), lambda i: (0, i, 0)),
                  pl.BlockSpec((NC, B, 1), lambda i: (0, i, 0))],
        out_specs=[pl.BlockSpec((B, D_IN), lambda i: (i, 0)),
                   pl.BlockSpec((B, 1), lambda i: (i, 0)),
                   pl.BlockSpec((B, 1), lambda i: (i, 0))],
        out_shape=[jax.ShapeDtypeStruct((N, D_IN), jnp.float32),
                   jax.ShapeDtypeStruct((N, 1), jnp.float32),
                   jax.ShapeDtypeStruct((N, 1), jnp.float32)],
    )(x, degs3, degd3)

    mesh = plsc.VectorSubcoreMesh(**_MESH)
    agg1 = pl.kernel(
        _agg1_body,
        out_type=jax.ShapeDtypeStruct((NC, NP, D_H), jnp.float32),
        mesh=mesh,
        scratch_types=(
            pltpu.VMEM((5, 1, 2, K), jnp.int32),
            pltpu.VMEM((3, K, D_H), jnp.float32),
            pltpu.VMEM_SHARED((NP, D_H), jnp.float32),
        ) + (pltpu.SemaphoreType.DMA,) * 8,
    )(xs, ei_t, zeros_h)

    b1r = b1.reshape(1, D_H)
    y2 = pl.pallas_call(
        _mlp_body,
        grid=(G,),
        in_specs=[pl.BlockSpec((NC, B, D_H), lambda i: (0, i, 0)),
                  pl.BlockSpec((B, 1), lambda i: (i, 0)),
                  pl.BlockSpec((B, 1), lambda i: (i, 0)),
                  pl.BlockSpec((D_H, D_H), lambda i: (0, 0)),
                  pl.BlockSpec((1, D_H), lambda i: (0, 0)),
                  pl.BlockSpec((D_H, D_OUT), lambda i: (0, 0))],
        out_specs=pl.BlockSpec((B, D_OUT), lambda i: (i, 0)),
        out_shape=jax.ShapeDtypeStruct((N, D_OUT), jnp.float32),
    )(agg1, nd, ns, W1, b1r, W2)

    mesh = plsc.VectorSubcoreMesh(**_MESH)
    agg2 = pl.kernel(
        _agg2_body,
        out_type=jax.ShapeDtypeStruct((NC, NP, D_OUT), jnp.float32),
        mesh=mesh,
        compiler_params=pltpu.CompilerParams(use_tc_tiling_on_sc=False),
        scratch_types=(
            pltpu.VMEM((NCH, 2, K), jnp.int32),
            pltpu.VMEM((NB2, K, D_OUT), jnp.float32),
            pltpu.VMEM_SHARED((NP, D_OUT), jnp.float32),
        ) + (pltpu.SemaphoreType.DMA,) * NB2,
    )(y2, ei_t, zeros_o)

    b2r = b2.reshape(1, D_OUT)
    out = pl.pallas_call(
        _final_body,
        grid=(G,),
        in_specs=[pl.BlockSpec((NC, B, D_OUT), lambda i: (0, i, 0)),
                  pl.BlockSpec((B, 1), lambda i: (i, 0)),
                  pl.BlockSpec((1, D_OUT), lambda i: (0, 0))],
        out_specs=pl.BlockSpec((B, D_OUT), lambda i: (i, 0)),
        out_shape=jax.ShapeDtypeStruct((N, D_OUT), jnp.float32),
    )(agg2, nd, b2r)
    return out


# speedup vs baseline: 1.0622x; 1.0103x over previous
import jax
import jax.numpy as jnp
from jax import lax
from jax.experimental import pallas as pl
from jax.experimental.pallas import tpu as pltpu
from jax.experimental.pallas import tpu_sc as plsc

N = 10000
E = 320000
D_IN = 128
D_H = 128
D_OUT = 16

NC = 2
NS = 16
NW = NC * NS
EPW = E // NW
K = 80
NCH = EPW // K
NP = 10240
RPT = NP // NS

_MESH = dict(core_axis_name="c", subcore_axis_name="s",
             num_cores=NC, num_subcores=NS)


NBUF = 5


def _deg_body(ei_hbm, zeros_hbm, ones_hbm, degs_out, degd_out,
              idx_v, ones_v, degs_sh, degd_sh, *sems):
    cid = lax.axis_index("c")
    sid = lax.axis_index("s")
    wid = cid * NS + sid
    pltpu.sync_copy(zeros_hbm, degs_sh.at[pl.ds(sid * RPT, RPT)])
    pltpu.sync_copy(zeros_hbm, degd_sh.at[pl.ds(sid * RPT, RPT)])
    pltpu.sync_copy(ones_hbm, ones_v)
    pltpu.sync_copy(ei_hbm.at[wid], idx_v)
    plsc.subcore_barrier()


    def _g(g, c):
        for b in range(NBUF):
            j = g * NBUF + b
            pltpu.async_copy(ones_v, degs_sh.at[idx_v.at[j, 0]], sems[b],
                             add=True)
            pltpu.async_copy(ones_v, degd_sh.at[idx_v.at[j, 1]],
                             sems[NBUF + b], add=True)
        for b in range(NBUF):
            j = g * NBUF + b
            pltpu.make_async_copy(ones_v, degs_sh.at[idx_v.at[j, 0]],
                                  sems[b]).wait()
            pltpu.make_async_copy(ones_v, degd_sh.at[idx_v.at[j, 1]],
                                  sems[NBUF + b]).wait()
        return c
    lax.fori_loop(0, NCH // NBUF, _g, 0)

    plsc.subcore_barrier()
    pltpu.sync_copy(degs_sh.at[pl.ds(sid * RPT, RPT)],
                    degs_out.at[cid, pl.ds(sid * RPT, RPT)])
    pltpu.sync_copy(degd_sh.at[pl.ds(sid * RPT, RPT)],
                    degd_out.at[cid, pl.ds(sid * RPT, RPT)])


def _agg1_body(x_hbm, ei_hbm, zeros_hbm, out_hbm,
               idx_v, rows_v, acc_sh, *sems):
    isems = sems[:5]
    gsems = sems[5:]
    cid = lax.axis_index("c")
    sid = lax.axis_index("s")
    wid = cid * NS + sid
    pltpu.sync_copy(zeros_hbm, acc_sh.at[pl.ds(sid * RPT, RPT), :])
    plsc.subcore_barrier()

    def idx_load(j, b5):
        pltpu.async_copy(ei_hbm.at[wid, pl.ds(j, 1)], idx_v.at[b5], isems[b5])

    def idx_wait(j, b5):
        pltpu.make_async_copy(ei_hbm.at[wid, pl.ds(j, 1)], idx_v.at[b5],
                              isems[b5]).wait()

    def gather(g, b5, b3):
        idx_wait(g, b5)
        pltpu.async_copy(x_hbm.at[idx_v.at[b5, 0, 0]], rows_v.at[b3],
                         gsems[b3])

    def step(j, i, do_load, do_gather):
        b3, b5 = i % 3, i % 5
        pltpu.make_async_copy(x_hbm.at[idx_v.at[b5, 0, 0]], rows_v.at[b3],
                              gsems[b3]).wait()
        pltpu.sync_copy(rows_v.at[b3], acc_sh.at[idx_v.at[b5, 0, 1]],
                        add=True)
        if do_load:
            idx_load(j + 5, b5)
        if do_gather:
            gather(j + 3, (i + 3) % 5, (i + 3) % 3)

    for b in range(5):
        idx_load(b, b)
    for c in range(3):
        gather(c, c, c)

    def _t(t, c):
        for i in range(15):
            j = t * 15 + i
            step(j, i, True, True)
        return c
    lax.fori_loop(0, NCH // 15, _t, 0)
    for j in range(NCH // 15 * 15, NCH):
        step(j, j % 15, j + 5 < NCH, j + 3 < NCH)

    plsc.subcore_barrier()
    pltpu.sync_copy(acc_sh.at[pl.ds(sid * RPT, RPT), :],
                    out_hbm.at[cid, pl.ds(sid * RPT, RPT), :])


NB2 = 10


def _agg2_body(x_hbm, ei_hbm, zeros_hbm, out_hbm,
               idx_v, rows_v, acc_sh, *sems):
    cid = lax.axis_index("c")
    sid = lax.axis_index("s")
    wid = cid * NS + sid
    pltpu.sync_copy(zeros_hbm, acc_sh.at[pl.ds(sid * RPT, RPT), :])
    pltpu.sync_copy(ei_hbm.at[wid], idx_v)
    plsc.subcore_barrier()

    for b in range(NB2):
        pltpu.async_copy(x_hbm.at[idx_v.at[b, 0]], rows_v.at[b], sems[b])

    def step(j, b, do_gather):
        pltpu.make_async_copy(x_hbm.at[idx_v.at[j, 0]], rows_v.at[b],
                              sems[b]).wait()
        pltpu.sync_copy(rows_v.at[b], acc_sh.at[idx_v.at[j, 1]], add=True)
        if do_gather:
            pltpu.async_copy(x_hbm.at[idx_v.at[j + NB2, 0]], rows_v.at[b],
                             sems[b])

    def _g(g, c):
        for b in range(NB2):
            step(g * NB2 + b, b, True)
        return c
    nfull = NCH // NB2 - 1
    lax.fori_loop(0, nfull, _g, 0)
    for j in range(nfull * NB2, NCH):
        step(j, j % NB2, j + NB2 < NCH)

    plsc.subcore_barrier()
    pltpu.sync_copy(acc_sh.at[pl.ds(sid * RPT, RPT), :],
                    out_hbm.at[cid, pl.ds(sid * RPT, RPT), :])


def _prep_body(x_ref, ds_ref, dd_ref, xs_ref, ns_ref, nd_ref):
    ns = lax.rsqrt(jnp.maximum(ds_ref[0] + ds_ref[1], 1.0))
    nd = lax.rsqrt(jnp.maximum(dd_ref[0] + dd_ref[1], 1.0))
    xs_ref[...] = x_ref[...] * ns
    ns_ref[...] = ns
    nd_ref[...] = nd


def _mlp_body(a0_ref, nd_ref, ns_ref, w1_ref, b1_ref, w2_ref, out_ref):
    a = (a0_ref[0] + a0_ref[1]) * nd_ref[...]
    h = jnp.dot(a, w1_ref[...], preferred_element_type=jnp.float32)
    h = jnp.maximum(h + b1_ref[...], 0.0)
    out_ref[...] = jnp.dot(h * ns_ref[...], w2_ref[...], preferred_element_type=jnp.float32)


def _final_body(a_ref, nd_ref, b2_ref, out_ref):
    out_ref[...] = (a_ref[0] + a_ref[1]) * nd_ref[...] + b2_ref[...]


def kernel(x, edge_index, W1, b1, W2, b2):
    ei_t = edge_index.reshape(2, NW, NCH, K).transpose(1, 2, 0, 3)
    zeros_r = jnp.zeros((RPT,), jnp.float32)
    ones_c = jnp.ones((K,), jnp.float32)
    zeros_h = jnp.zeros((RPT, D_H), jnp.float32)
    zeros_o = jnp.zeros((RPT, D_OUT), jnp.float32)

    mesh = plsc.VectorSubcoreMesh(**_MESH)
    degs_p, degd_p = pl.kernel(
        _deg_body,
        out_type=(jax.ShapeDtypeStruct((NC, NP), jnp.float32),
                  jax.ShapeDtypeStruct((NC, NP), jnp.float32)),
        mesh=mesh,
        scratch_types=(
            pltpu.VMEM((NCH, 2, K), jnp.int32),
            pltpu.VMEM((K,), jnp.float32),
            pltpu.VMEM_SHARED((NP,), jnp.float32),
            pltpu.VMEM_SHARED((NP,), jnp.float32),
        ) + (pltpu.SemaphoreType.DMA,) * (2 * NBUF),
    )(ei_t, zeros_r, ones_c)

    B = 5000
    G = N // B
    degs3 = degs_p.reshape(NC, NP, 1)
    degd3 = degd_p.reshape(NC, NP, 1)
    xs, ns, nd = pl.pallas_call(
        _prep_body,
        grid=(G,),
        in_specs=[pl.BlockSpec((B, D_IN), lambda i: (i, 0)),
                  pl.BlockSpec((NC, B, 1), lambda i: (0, i, 0)),
                  pl.BlockSpec((NC, B, 1), lambda i: (0, i, 0))],
        out_specs=[pl.BlockSpec((B, D_IN), lambda i: (i, 0)),
                   pl.BlockSpec((B, 1), lambda i: (i, 0)),
                   pl.BlockSpec((B, 1), lambda i: (i, 0))],
        out_shape=[jax.ShapeDtypeStruct((N, D_IN), jnp.float32),
                   jax.ShapeDtypeStruct((N, 1), jnp.float32),
                   jax.ShapeDtypeStruct((N, 1), jnp.float32)],
    )(x, degs3, degd3)

    mesh = plsc.VectorSubcoreMesh(**_MESH)
    agg1 = pl.kernel(
        _agg1_body,
        out_type=jax.ShapeDtypeStruct((NC, NP, D_H), jnp.float32),
        mesh=mesh,
        scratch_types=(
            pltpu.VMEM((5, 1, 2, K), jnp.int32),
            pltpu.VMEM((3, K, D_H), jnp.float32),
            pltpu.VMEM_SHARED((NP, D_H), jnp.float32),
        ) + (pltpu.SemaphoreType.DMA,) * 8,
    )(xs, ei_t, zeros_h)

    b1r = b1.reshape(1, D_H)
    y2 = pl.pallas_call(
        _mlp_body,
        grid=(G,),
        in_specs=[pl.BlockSpec((NC, B, D_H), lambda i: (0, i, 0)),
                  pl.BlockSpec((B, 1), lambda i: (i, 0)),
                  pl.BlockSpec((B, 1), lambda i: (i, 0)),
                  pl.BlockSpec((D_H, D_H), lambda i: (0, 0)),
                  pl.BlockSpec((1, D_H), lambda i: (0, 0)),
                  pl.BlockSpec((D_H, D_OUT), lambda i: (0, 0))],
        out_specs=pl.BlockSpec((B, D_OUT), lambda i: (i, 0)),
        out_shape=jax.ShapeDtypeStruct((N, D_OUT), jnp.float32),
    )(agg1, nd, ns, W1, b1r, W2)

    mesh = plsc.VectorSubcoreMesh(**_MESH)
    agg2 = pl.kernel(
        _agg2_body,
        out_type=jax.ShapeDtypeStruct((NC, NP, D_OUT), jnp.float32),
        mesh=mesh,
        compiler_params=pltpu.CompilerParams(use_tc_tiling_on_sc=False),
        scratch_types=(
            pltpu.VMEM((NCH, 2, K), jnp.int32),
            pltpu.VMEM((NB2, K, D_OUT), jnp.float32),
            pltpu.VMEM_SHARED((NP, D_OUT), jnp.float32),
        ) + (pltpu.SemaphoreType.DMA,) * NB2,
    )(y2, ei_t, zeros_o)

    b2r = b2.reshape(1, D_OUT)
    out = pl.pallas_call(
        _final_body,
        grid=(G,),
        in_specs=[pl.BlockSpec((NC, B, D_OUT), lambda i: (0, i, 0)),
                  pl.BlockSpec((B, 1), lambda i: (i, 0)),
                  pl.BlockSpec((1, D_OUT), lambda i: (0, 0))],
        out_specs=pl.BlockSpec((B, D_OUT), lambda i: (i, 0)),
        out_shape=jax.ShapeDtypeStruct((N, D_OUT), jnp.float32),
    )(agg2, nd, b2r)
    return out
